# Initial kernel scaffold; baseline (speedup 1.0000x reference)
import jax
import jax.numpy as jnp
from jax.experimental import pallas as pl


def kernel(x):
    raise NotImplementedError("write your pallas kernel here")


# speedup vs baseline: 5.5966x; 5.5966x over previous
import jax
import jax.numpy as jnp
from jax.experimental import pallas as pl


def _bitonic_merge(a, desc):
    n = a.shape[0]
    if n == 1:
        return a
    h = n // 2
    x, y = a[:h], a[h:]
    hi = jnp.maximum(x, y)
    lo = jnp.minimum(x, y)
    first, second = (hi, lo) if desc else (lo, hi)
    return jnp.concatenate(
        [_bitonic_merge(first, desc), _bitonic_merge(second, desc)], axis=0
    )


def _bitonic_sort(a, desc):
    n = a.shape[0]
    if n == 1:
        return a
    h = n // 2
    lo = _bitonic_sort(a[:h], True)
    hi = _bitonic_sort(a[h:], False)
    return _bitonic_merge(jnp.concatenate([lo, hi], axis=0), desc)


def _body(x_ref, o_ref):
    v = x_ref[0]
    y = jnp.maximum(v, 0.0)
    a = _bitonic_sort(y[:64], True)
    b = _bitonic_sort(y[64:96], True)
    bf = jnp.concatenate([b[i:i + 1] for i in range(31, -1, -1)], axis=0)
    cand = jnp.minimum(a[15:47], bf)
    t = jnp.maximum(jnp.max(cand, axis=0), a[47])
    o_ref[0] = jnp.where(y >= t[None], y, 0.0)


def kernel(x):
    B, C, H, W = x.shape
    assert C == 96
    HW = H * W
    LANES = 128
    assert HW % LANES == 0
    nrow = HW // LANES
    ROWS = 8 if nrow % 8 == 0 else 1
    xr = x.reshape(B, C, nrow, LANES)
    grid = (B, nrow // ROWS)
    out = pl.pallas_call(
        _body,
        grid=grid,
        in_specs=[pl.BlockSpec((1, C, ROWS, LANES), lambda b, r: (b, 0, r, 0))],
        out_specs=pl.BlockSpec((1, C, ROWS, LANES), lambda b, r: (b, 0, r, 0)),
        out_shape=jax.ShapeDtypeStruct((B, C, nrow, LANES), x.dtype),
    )(xr)
    return out.reshape(B, C, H, W)


# speedup vs baseline: 7.3395x; 1.3114x over previous
import jax
import jax.numpy as jnp
from jax.experimental import pallas as pl


def _bitonic_merge(a, desc):
    n = a.shape[0]
    if n == 1:
        return a
    h = n // 2
    x, y = a[:h], a[h:]
    hi = jnp.maximum(x, y)
    lo = jnp.minimum(x, y)
    first, second = (hi, lo) if desc else (lo, hi)
    return jnp.concatenate(
        [_bitonic_merge(first, desc), _bitonic_merge(second, desc)], axis=0
    )


def _bitonic_sort(a, desc):
    n = a.shape[0]
    if n == 1:
        return a
    h = n // 2
    lo = _bitonic_sort(a[:h], True)
    hi = _bitonic_sort(a[h:], False)
    return _bitonic_merge(jnp.concatenate([lo, hi], axis=0), desc)


def _body(x_ref, o_ref):
    v = x_ref[0]
    y = jnp.maximum(v, 0.0)
    a = _bitonic_sort(y[:64], True)
    b = _bitonic_sort(y[64:96], True)
    bf = jnp.concatenate([b[i:i + 1] for i in range(31, -1, -1)], axis=0)
    cand = jnp.minimum(a[15:47], bf)
    t = jnp.maximum(jnp.max(cand, axis=0), a[47])
    o_ref[0] = jnp.where(y >= t[None], y, 0.0)


def kernel(x):
    B, C, H, W = x.shape
    assert C == 96
    HW = H * W
    LANES = 128
    assert HW % LANES == 0
    nrow = HW // LANES
    ROWS = 56 if nrow % 56 == 0 else (8 if nrow % 8 == 0 else 1)
    xr = x.reshape(B, C, nrow, LANES)
    grid = (B, nrow // ROWS)
    out = pl.pallas_call(
        _body,
        grid=grid,
        in_specs=[pl.BlockSpec((1, C, ROWS, LANES), lambda b, r: (b, 0, r, 0))],
        out_specs=pl.BlockSpec((1, C, ROWS, LANES), lambda b, r: (b, 0, r, 0)),
        out_shape=jax.ShapeDtypeStruct((B, C, nrow, LANES), x.dtype),
    )(xr)
    return out.reshape(B, C, H, W)
